# Initial kernel scaffold; baseline (speedup 1.0000x reference)
import functools

import jax
import jax.numpy as jnp
from jax import lax
from jax.experimental import pallas as pl
from jax.experimental.pallas import tpu as pltpu
from jax.experimental.pallas import tpu_sc as plsc

_S = 50
_C = 20
_D = 64
_B = 1024
_NW = 32
_NV = _B * _S
_VPW = _NV // _NW
_G = 32
_NCH = _VPW // _G
_RPC = _G * _C
_SUB = 128
_NSUB = _RPC // _SUB


def _sc_body(codes_hbm, lens_hbm, table_hbm, out_hbm,
             idx_v, rows_v, out_v, lens_v, recip_v, n0_v, t0_v, sem):
    nc = 2
    wid = lax.axis_index("s") * nc + lax.axis_index("c")
    base = wid * _VPW

    pltpu.sync_copy(lens_hbm, lens_v)
    pltpu.sync_copy(table_hbm.at[pl.ds(0, 1)], t0_v)
    t0 = [t0_v[0, pl.ds(k * 16, 16)] for k in range(_D // 16)]
    iota = jnp.arange(16, dtype=jnp.int32)

    def chunk(ci, carry):
        off = base + ci * _G
        pltpu.sync_copy(codes_hbm.at[pl.ds(off * _C, _RPC)], idx_v)
        copies = [
            pltpu.async_copy(
                table_hbm.at[idx_v.at[pl.ds(j * _SUB, _SUB)]],
                rows_v.at[pl.ds(j * _SUB, _SUB)],
                sem,
            )
            for j in range(_NSUB)
        ]
        for cp in copies:
            cp.wait()

        for g in range(_G // 16):
            lvis = g * 16 + iota
            gvis = off + lvis
            b = gvis // _S
            s = gvis - b * _S
            ln = plsc.load_gather(lens_v, [b])
            valid = s < ln
            cnt = jnp.zeros((16,), jnp.int32)
            for c in range(_C):
                code = plsc.load_gather(idx_v, [lvis * _C + c])
                cnt = cnt + (code > 0).astype(jnp.int32)
            cntf = cnt.astype(jnp.float32)
            recip = jnp.where(valid & (cnt > 0), 1.0 / cntf, 0.0)
            n0 = (_C - cnt).astype(jnp.float32)
            recip_v[pl.ds(g * 16, 16)] = recip
            n0_v[pl.ds(g * 16, 16)] = n0

        def visit(v, c2):
            accs = [jnp.zeros((16,), jnp.float32) for _ in range(_D // 16)]
            r0 = v * _C
            for c in range(_C):
                for k in range(_D // 16):
                    accs[k] = accs[k] + rows_v[r0 + c, pl.ds(k * 16, 16)]
            vv = jnp.full((16,), v, jnp.int32)
            rec = plsc.load_gather(recip_v, [vv])
            n0 = plsc.load_gather(n0_v, [vv])
            for k in range(_D // 16):
                out_v[v, pl.ds(k * 16, 16)] = (accs[k] - n0 * t0[k]) * rec
            return c2

        lax.fori_loop(0, _G, visit, 0)
        pltpu.sync_copy(out_v, out_hbm.at[pl.ds(off, _G)])
        return carry

    lax.fori_loop(0, _NCH, chunk, 0)


_sc_call = functools.partial(
    pl.kernel,
    out_type=jax.ShapeDtypeStruct((_NV, _D), jnp.float32),
    mesh=plsc.VectorSubcoreMesh(core_axis_name="c", subcore_axis_name="s"),
    scratch_types=[
        pltpu.VMEM((_RPC,), jnp.int32),
        pltpu.VMEM((_RPC, _D), jnp.float32),
        pltpu.VMEM((_G, _D), jnp.float32),
        pltpu.VMEM((_B,), jnp.int32),
        pltpu.VMEM((_G,), jnp.float32),
        pltpu.VMEM((_G,), jnp.float32),
        pltpu.VMEM((1, _D), jnp.float32),
        pltpu.SemaphoreType.DMA,
    ],
)(_sc_body)


@jax.jit
def kernel(code_embeddings, visit_codes, visit_lens):
    codes_flat = visit_codes.reshape(-1)
    out = _sc_call(codes_flat, visit_lens, code_embeddings)
    return out.reshape(_B, _S, _D)


# speedup vs baseline: 11.5282x; 11.5282x over previous
import functools

import jax
import jax.numpy as jnp
from jax import lax
from jax.experimental import pallas as pl
from jax.experimental.pallas import tpu as pltpu
from jax.experimental.pallas import tpu_sc as plsc

_S = 50
_C = 20
_D = 64
_B = 1024
_NW = 32
_NV = _B * _S
_VPW = _NV // _NW
_G = 32
_NCH = _VPW // _G
_RPC = _G * _C
_SUB = 128
_NSUB = _RPC // _SUB


def _sc_body(codes_hbm, lens_hbm, table_hbm, out_hbm,
             idx_v, rows_v, out_v, lens_v, recip_v, n0_v, t0_v, sem):
    nc = 2
    wid = lax.axis_index("s") * nc + lax.axis_index("c")
    base = wid * _VPW

    pltpu.sync_copy(lens_hbm, lens_v)
    pltpu.sync_copy(table_hbm.at[pl.ds(0, 1)], t0_v)
    t0 = [t0_v[0, pl.ds(k * 16, 16)] for k in range(_D // 16)]
    iota = jnp.arange(16, dtype=jnp.int32)

    def chunk(ci, carry):
        off = base + ci * _G
        pltpu.sync_copy(codes_hbm.at[pl.ds(off * _C, _RPC)], idx_v)
        copies = [
            pltpu.async_copy(
                table_hbm.at[idx_v.at[pl.ds(j * _SUB, _SUB)]],
                rows_v.at[pl.ds(j * _SUB, _SUB)],
                sem,
            )
            for j in range(_NSUB)
        ]
        for cp in copies:
            cp.wait()

        for g in range(_G // 16):
            lvis = g * 16 + iota
            gvis = off + lvis
            b = gvis // _S
            s = gvis - b * _S
            ln = plsc.load_gather(lens_v, [b])
            valid = s < ln
            cnt = jnp.zeros((16,), jnp.int32)
            for c in range(_C):
                code = plsc.load_gather(idx_v, [lvis * _C + c])
                cnt = cnt + (code > 0).astype(jnp.int32)
            cntf = cnt.astype(jnp.float32)
            recip = jnp.where(valid & (cnt > 0), 1.0 / cntf, 0.0)
            n0 = (_C - cnt).astype(jnp.float32)
            recip_v[pl.ds(g * 16, 16)] = recip
            n0_v[pl.ds(g * 16, 16)] = n0

        def visit(v, c2):
            accs = [jnp.zeros((16,), jnp.float32) for _ in range(_D // 16)]
            r0 = v * _C
            for c in range(_C):
                for k in range(_D // 16):
                    accs[k] = accs[k] + rows_v[r0 + c, pl.ds(k * 16, 16)]
            vv = jnp.full((16,), v, jnp.int32)
            rec = plsc.load_gather(recip_v, [vv])
            n0 = plsc.load_gather(n0_v, [vv])
            for k in range(_D // 16):
                out_v[v, pl.ds(k * 16, 16)] = (accs[k] - n0 * t0[k]) * rec
            return c2

        lax.fori_loop(0, _G, visit, 0)
        pltpu.sync_copy(out_v, out_hbm.at[pl.ds(off, _G)])
        return carry

    lax.fori_loop(0, _NCH, chunk, 0)


_sc_call = functools.partial(
    pl.kernel,
    out_type=jax.ShapeDtypeStruct((_NV, _D), jnp.float32),
    mesh=plsc.VectorSubcoreMesh(core_axis_name="c", subcore_axis_name="s"),
    scratch_types=[
        pltpu.VMEM((_RPC,), jnp.int32),
        pltpu.VMEM((_RPC, _D), jnp.float32),
        pltpu.VMEM((_G, _D), jnp.float32),
        pltpu.VMEM((_B,), jnp.int32),
        pltpu.VMEM((_G,), jnp.float32),
        pltpu.VMEM((_G,), jnp.float32),
        pltpu.VMEM((1, _D), jnp.float32),
        pltpu.SemaphoreType.DMA,
    ],
    compiler_params=pltpu.CompilerParams(
        use_tc_tiling_on_sc=False, needs_layout_passes=False
    ),
)(_sc_body)


@jax.jit
def kernel(code_embeddings, visit_codes, visit_lens):
    codes_flat = visit_codes.reshape(-1)
    out = _sc_call(codes_flat, visit_lens, code_embeddings)
    return out.reshape(_B, _S, _D)


# speedup vs baseline: 15.8361x; 1.3737x over previous
import functools

import jax
import jax.numpy as jnp
from jax import lax
from jax.experimental import pallas as pl
from jax.experimental.pallas import tpu as pltpu
from jax.experimental.pallas import tpu_sc as plsc

_S = 50
_C = 20
_D = 64
_B = 1024
_NW = 32
_NV = _B * _S
_VPW = _NV // _NW
_G = 32
_NCH = _VPW // _G
_RPC = _G * _C
_SUB = 128
_NSUB = _RPC // _SUB


def _sc_body(codes_hbm, lens_hbm, table_hbm, out_hbm,
             idx_v, rows_v, out_v, lens_v, recip_v, n0_v, t0_v,
             sem0, sem1, osem):
    nc = 2
    wid = lax.axis_index("s") * nc + lax.axis_index("c")
    base = wid * _VPW

    pltpu.sync_copy(lens_hbm, lens_v)
    pltpu.sync_copy(table_hbm.at[pl.ds(0, 1)], t0_v)
    t0 = [t0_v[0, pl.ds(k * 16, 16)] for k in range(_D // 16)]
    iota = jnp.arange(16, dtype=jnp.int32)
    sems = (sem0, sem1)

    def stage_and_fire(ci, slot):
        off = base + ci * _G
        pltpu.sync_copy(codes_hbm.at[pl.ds(off * _C, _RPC)], idx_v.at[slot])
        for j in range(_NSUB):
            pltpu.async_copy(
                table_hbm.at[idx_v.at[slot].at[pl.ds(j * _SUB, _SUB)]],
                rows_v.at[slot].at[pl.ds(j * _SUB, _SUB)],
                sems[slot],
            )

    def drain(ci, slot):
        for j in range(_NSUB):
            pltpu.make_async_copy(
                table_hbm.at[idx_v.at[slot].at[pl.ds(j * _SUB, _SUB)]],
                rows_v.at[slot].at[pl.ds(j * _SUB, _SUB)],
                sems[slot],
            ).wait()

    def counts(ci, slot):
        off = base + ci * _G
        for g in range(_G // 16):
            lvis = g * 16 + iota
            gvis = off + lvis
            b = gvis // _S
            s = gvis - b * _S
            ln = plsc.load_gather(lens_v, [b])
            valid = s < ln
            cnt = jnp.zeros((16,), jnp.int32)
            for c in range(_C):
                code = plsc.load_gather(idx_v.at[slot], [lvis * _C + c])
                cnt = cnt + (code > 0).astype(jnp.int32)
            cntf = cnt.astype(jnp.float32)
            recip = jnp.where(valid & (cnt > 0), 1.0 / cntf, 0.0)
            n0 = (_C - cnt).astype(jnp.float32)
            recip_v[pl.ds(g * 16, 16)] = recip
            n0_v[pl.ds(g * 16, 16)] = n0

    def visits_and_store(ci, slot):
        def visit(v, c2):
            accs = [jnp.zeros((16,), jnp.float32) for _ in range(_D // 16)]
            r0 = v * _C
            for c in range(_C):
                for k in range(_D // 16):
                    accs[k] = accs[k] + rows_v[slot, r0 + c, pl.ds(k * 16, 16)]
            vv = jnp.full((16,), v, jnp.int32)
            rec = plsc.load_gather(recip_v, [vv])
            n0 = plsc.load_gather(n0_v, [vv])
            for k in range(_D // 16):
                out_v[slot, v, pl.ds(k * 16, 16)] = (accs[k] - n0 * t0[k]) * rec
            return c2

        lax.fori_loop(0, _G, visit, 0)
        off = base + ci * _G
        pltpu.sync_copy(out_v.at[slot], out_hbm.at[pl.ds(off, _G)])

    stage_and_fire(0, 0)

    def pair(i, carry):
        c0 = 2 * i
        stage_and_fire(c0 + 1, 1)
        counts(c0, 0)
        drain(c0, 0)
        visits_and_store(c0, 0)

        @pl.when(c0 + 2 < _NCH)
        def _():
            stage_and_fire(c0 + 2, 0)

        counts(c0 + 1, 1)
        drain(c0 + 1, 1)
        visits_and_store(c0 + 1, 1)
        return carry

    lax.fori_loop(0, _NCH // 2, pair, 0)


_sc_call = functools.partial(
    pl.kernel,
    out_type=jax.ShapeDtypeStruct((_NV, _D), jnp.float32),
    mesh=plsc.VectorSubcoreMesh(core_axis_name="c", subcore_axis_name="s"),
    scratch_types=[
        pltpu.VMEM((2, _RPC), jnp.int32),
        pltpu.VMEM((2, _RPC, _D), jnp.float32),
        pltpu.VMEM((2, _G, _D), jnp.float32),
        pltpu.VMEM((_B,), jnp.int32),
        pltpu.VMEM((_G,), jnp.float32),
        pltpu.VMEM((_G,), jnp.float32),
        pltpu.VMEM((1, _D), jnp.float32),
        pltpu.SemaphoreType.DMA,
        pltpu.SemaphoreType.DMA,
        pltpu.SemaphoreType.DMA,
    ],
    compiler_params=pltpu.CompilerParams(
        use_tc_tiling_on_sc=False, needs_layout_passes=False
    ),
)(_sc_body)


@jax.jit
def kernel(code_embeddings, visit_codes, visit_lens):
    codes_flat = visit_codes.reshape(-1)
    out = _sc_call(codes_flat, visit_lens, code_embeddings)
    return out.reshape(_B, _S, _D)


# speedup vs baseline: 15.8641x; 1.0018x over previous
import functools

import jax
import jax.numpy as jnp
from jax import lax
from jax.experimental import pallas as pl
from jax.experimental.pallas import tpu as pltpu
from jax.experimental.pallas import tpu_sc as plsc

_S = 50
_C = 20
_D = 64
_B = 1024
_NW = 32
_NV = _B * _S
_RPW = _B // _NW
_SUB = 128
_S0 = (0, 16, 32)
_SZ = (16, 16, 18)
_NK = 4


def _splits(n_rows):
    out, off = [], 0
    while off < n_rows:
        n = min(_SUB, n_rows - off)
        out.append((off, n))
        off += n
    return out


def _sc_body(codes_hbm, lens_hbm, table_hbm, out_hbm,
             idx_v, rows_v, out_v, zero_v, lens_v, recip_v, n0_v, t0_v,
             sem0, sem1, sem2):
    nc = 2
    wid = lax.axis_index("s") * nc + lax.axis_index("c")
    b0 = wid * _RPW

    pltpu.sync_copy(lens_hbm, lens_v)
    pltpu.sync_copy(table_hbm.at[pl.ds(0, 1)], t0_v)
    t0 = [t0_v[0, pl.ds(k * 16, 16)] for k in range(_NK)]
    iota = jnp.arange(16, dtype=jnp.int32)
    zf = jnp.zeros((16,), jnp.float32)
    for v in range(_SZ[2]):
        for k in range(_NK):
            zero_v[v, pl.ds(k * 16, 16)] = zf
    sems = (sem0, sem1, sem2)

    def fire(b, t):
        s0, sz = _S0[t], _SZ[t]
        pltpu.sync_copy(codes_hbm.at[pl.ds((b * _S + s0) * _C, sz * _C)],
                        idx_v.at[t].at[pl.ds(0, sz * _C)])
        for off, n in _splits(sz * _C):
            pltpu.async_copy(
                table_hbm.at[idx_v.at[t].at[pl.ds(off, n)]],
                rows_v.at[t].at[pl.ds(off, n)],
                sems[t],
            )

    def drain(t):
        sz = _SZ[t]
        for off, n in _splits(sz * _C):
            pltpu.make_async_copy(
                table_hbm.at[idx_v.at[t].at[pl.ds(off, n)]],
                rows_v.at[t].at[pl.ds(off, n)],
                sems[t],
            ).wait()

    def counts(t, lnv):
        s0, sz = _S0[t], _SZ[t]
        for g in range((sz + 15) // 16):
            lvis = g * 16 + iota
            valid = (s0 + lvis) < lnv
            cnt = jnp.zeros((16,), jnp.int32)
            for c in range(_C):
                code = plsc.load_gather(idx_v.at[t], [lvis * _C + c])
                cnt = cnt + (code > 0).astype(jnp.int32)
            cntf = cnt.astype(jnp.float32)
            recip = jnp.where(valid & (cnt > 0), 1.0 / cntf, 0.0)
            n0 = (_C - cnt).astype(jnp.float32)
            recip_v[pl.ds(g * 16, 16)] = recip
            n0_v[pl.ds(g * 16, 16)] = n0

    def visits_and_store(b, t):
        s0, sz = _S0[t], _SZ[t]

        def visit(v, c2):
            accs = [jnp.zeros((16,), jnp.float32) for _ in range(_NK)]
            r0 = v * _C
            for c in range(_C):
                for k in range(_NK):
                    accs[k] = accs[k] + rows_v[t, r0 + c, pl.ds(k * 16, 16)]
            vv = jnp.full((16,), v, jnp.int32)
            rec = plsc.load_gather(recip_v, [vv])
            n0 = plsc.load_gather(n0_v, [vv])
            for k in range(_NK):
                out_v[t, v, pl.ds(k * 16, 16)] = (accs[k] - n0 * t0[k]) * rec
            return c2

        lax.fori_loop(0, sz, visit, 0)
        pltpu.sync_copy(out_v.at[t].at[pl.ds(0, sz)],
                        out_hbm.at[pl.ds(b * _S + s0, sz)])

    def zero_store(b, t):
        s0, sz = _S0[t], _SZ[t]
        pltpu.sync_copy(zero_v.at[pl.ds(0, sz)],
                        out_hbm.at[pl.ds(b * _S + s0, sz)])

    def cond_chunk(b, t, live):
        @pl.when(live)
        def _():
            counts(t, plsc.load_gather(lens_v, [jnp.full((16,), b, jnp.int32)]))
            drain(t)
            visits_and_store(b, t)

        @pl.when(jnp.logical_not(live))
        def _():
            zero_store(b, t)

    fire(b0, 0)

    def row(r, carry):
        b = b0 + r
        lnv = plsc.load_gather(lens_v, [jnp.full((16,), b, jnp.int32)])
        ln = jnp.max(lnv)

        live1 = ln > _S0[1]
        live2 = ln > _S0[2]

        @pl.when(live1)
        def _():
            fire(b, 1)

        counts(0, lnv)
        drain(0)
        visits_and_store(b, 0)

        @pl.when(live2)
        def _():
            fire(b, 2)

        cond_chunk(b, 1, live1)

        @pl.when(r + 1 < _RPW)
        def _():
            fire(b + 1, 0)

        cond_chunk(b, 2, live2)
        return carry

    lax.fori_loop(0, _RPW, row, 0)


_MAXR = _SZ[2] * _C
_IDXPAD = 640

_sc_call = functools.partial(
    pl.kernel,
    out_type=jax.ShapeDtypeStruct((_NV, _D), jnp.float32),
    mesh=plsc.VectorSubcoreMesh(core_axis_name="c", subcore_axis_name="s"),
    scratch_types=[
        pltpu.VMEM((3, _IDXPAD), jnp.int32),
        pltpu.VMEM((3, _MAXR, _D), jnp.float32),
        pltpu.VMEM((3, _SZ[2], _D), jnp.float32),
        pltpu.VMEM((_SZ[2], _D), jnp.float32),
        pltpu.VMEM((_B,), jnp.int32),
        pltpu.VMEM((32,), jnp.float32),
        pltpu.VMEM((32,), jnp.float32),
        pltpu.VMEM((1, _D), jnp.float32),
        pltpu.SemaphoreType.DMA,
        pltpu.SemaphoreType.DMA,
        pltpu.SemaphoreType.DMA,
    ],
    compiler_params=pltpu.CompilerParams(
        use_tc_tiling_on_sc=False, needs_layout_passes=False
    ),
)(_sc_body)


@jax.jit
def kernel(code_embeddings, visit_codes, visit_lens):
    codes_flat = visit_codes.reshape(-1)
    out = _sc_call(codes_flat, visit_lens, code_embeddings)
    return out.reshape(_B, _S, _D)
